# baseline (speedup 1.0000x reference)
import jax
import jax.numpy as jnp
from jax import lax
from jax.experimental import pallas as pl
from jax.experimental.pallas import tpu as pltpu

_NUM_CLASSES = 80
_ROWS = 2


def _decode_body(anch_ref, in_ref, out_ref):
    yp = pl.program_id(0)
    v = in_ref[...]
    g = v.shape[1]
    b = v.shape[2]
    c5 = _NUM_CLASSES + 5
    ctot = v.shape[3]

    lane = lax.broadcasted_iota(jnp.int32, (v.shape[0], g, b, ctot), 3)
    cmod = lane % c5
    xf = lax.broadcasted_iota(jnp.int32, (v.shape[0], g, b, ctot), 1).astype(jnp.float32)
    y2 = lax.broadcasted_iota(jnp.int32, (v.shape[0], g, b, ctot), 0)
    yf = (y2 + v.shape[0] * yp).astype(jnp.float32)
    off = jnp.where(cmod == 0, xf, yf)
    inv_g = 1.0 / g

    aw = jnp.where(lane < c5, anch_ref[0],
                   jnp.where(lane < 2 * c5, anch_ref[2], anch_ref[4]))
    ah = jnp.where(lane < c5, anch_ref[1],
                   jnp.where(lane < 2 * c5, anch_ref[3], anch_ref[5]))
    scale = jnp.where(cmod == 2, aw, ah)

    sig = jax.nn.sigmoid(v)
    ex = sig / (1.0 - sig)
    dec = jnp.where(
        cmod < 2, (sig + off) * inv_g,
        jnp.where(cmod < 4, ex * scale, sig))
    dect = jnp.transpose(dec.reshape(v.shape[0] * g, b, ctot), (1, 0, 2))
    for a in range(ctot // c5):
        out_ref[:, a] = dect[:, :, a * c5:(a + 1) * c5]


def kernel(inputs, anchors):
    B = inputs.shape[0]
    G = inputs.shape[2]
    A = anchors.shape[0]
    C5 = _NUM_CLASSES + 5
    N = G * G

    xt = jnp.transpose(inputs, (2, 3, 0, 1))
    anch_flat = anchors.reshape(-1)

    grid_spec = pltpu.PrefetchScalarGridSpec(
        num_scalar_prefetch=1,
        grid=(G // _ROWS,),
        in_specs=[
            pl.BlockSpec((_ROWS, G, B, A * C5), lambda yp, s: (yp, 0, 0, 0)),
        ],
        out_specs=pl.BlockSpec((B, A, _ROWS * G, C5), lambda yp, s: (0, 0, yp, 0)),
    )

    out = pl.pallas_call(
        _decode_body,
        grid_spec=grid_spec,
        out_shape=jax.ShapeDtypeStruct((B, A, N, C5), jnp.float32),
    )(anch_flat, xt)

    return out.reshape(B, A * N, C5)


# speedup vs baseline: 1.0488x; 1.0488x over previous
import jax
import jax.numpy as jnp
from jax import lax
from jax.experimental import pallas as pl
from jax.experimental.pallas import tpu as pltpu

_NUM_CLASSES = 80
_ROWS = 4


def _decode_body(anch_ref, in_ref, out_ref):
    yp = pl.program_id(0)
    v = in_ref[...]
    g = v.shape[1]
    b = v.shape[2]
    c5 = _NUM_CLASSES + 5
    ctot = v.shape[3]

    lane = lax.broadcasted_iota(jnp.int32, (v.shape[0], g, b, ctot), 3)
    cmod = lane % c5
    xf = lax.broadcasted_iota(jnp.int32, (v.shape[0], g, b, ctot), 1).astype(jnp.float32)
    y2 = lax.broadcasted_iota(jnp.int32, (v.shape[0], g, b, ctot), 0)
    yf = (y2 + v.shape[0] * yp).astype(jnp.float32)
    off = jnp.where(cmod == 0, xf, yf)
    inv_g = 1.0 / g

    aw = jnp.where(lane < c5, anch_ref[0],
                   jnp.where(lane < 2 * c5, anch_ref[2], anch_ref[4]))
    ah = jnp.where(lane < c5, anch_ref[1],
                   jnp.where(lane < 2 * c5, anch_ref[3], anch_ref[5]))
    scale = jnp.where(cmod == 2, aw, ah)

    sig = jax.nn.sigmoid(v)
    ex = sig / (1.0 - sig)
    dec = jnp.where(
        cmod < 2, (sig + off) * inv_g,
        jnp.where(cmod < 4, ex * scale, sig))
    dect = jnp.transpose(dec.reshape(v.shape[0] * g, b, ctot), (1, 0, 2))
    for a in range(ctot // c5):
        out_ref[:, a] = dect[:, :, a * c5:(a + 1) * c5]


def kernel(inputs, anchors):
    B = inputs.shape[0]
    G = inputs.shape[2]
    A = anchors.shape[0]
    C5 = _NUM_CLASSES + 5
    N = G * G

    xt = jnp.transpose(inputs, (2, 3, 0, 1))
    anch_flat = anchors.reshape(-1)

    grid_spec = pltpu.PrefetchScalarGridSpec(
        num_scalar_prefetch=1,
        grid=(G // _ROWS,),
        in_specs=[
            pl.BlockSpec((_ROWS, G, B, A * C5), lambda yp, s: (yp, 0, 0, 0)),
        ],
        out_specs=pl.BlockSpec((B, A, _ROWS * G, C5), lambda yp, s: (0, 0, yp, 0)),
    )

    out = pl.pallas_call(
        _decode_body,
        grid_spec=grid_spec,
        out_shape=jax.ShapeDtypeStruct((B, A, N, C5), jnp.float32),
    )(anch_flat, xt)

    return out.reshape(B, A * N, C5)
